# Initial kernel scaffold; baseline (speedup 1.0000x reference)
import jax
import jax.numpy as jnp
from jax.experimental import pallas as pl


def kernel(im, disp):
    raise NotImplementedError("write your pallas kernel here")


# speedup vs baseline: 92.3131x; 92.3131x over previous
import math

import jax
import jax.numpy as jnp
from jax import lax
from jax.experimental import pallas as pl
from jax.experimental.pallas import tpu as pltpu
from jax.experimental.pallas import tpu_sc as plsc

B, C, H, W = 4, 3, 512, 512
NC, NS, L = 2, 16, 16
NW = NC * NS
ROWS = B * H
RPW = ROWS // NW
TPB = H // RPW
RBLK = 8
NBLK = RPW // RBLK
PAD = 48
AW = 576
EPS = 1e-6
LN_BASE = math.log(1.414)


def _min_body(d_ref, o_ref):
    o_ref[...] = jnp.broadcast_to(jnp.min(d_ref[...]), (8, 128))


def _sc_body(im_hbm, disp_hbm, gmin_hbm, res_hbm, occ_hbm,
             disp_v, im_v, acc_v, res_v, occ_v, gmin_v):
    cid = lax.axis_index("c")
    sid = lax.axis_index("s")
    wid = sid * NC + cid
    b = wid // TPB
    y0 = (wid % TPB) * RPW

    pltpu.sync_copy(gmin_hbm.at[0], gmin_v)
    gmin = gmin_v[pl.ds(0, L)]
    lane_f = lax.iota(jnp.int32, L).astype(jnp.float32)

    def block_body(blk, carry):
        y = y0 + blk * RBLK
        pltpu.sync_copy(disp_hbm.at[b, pl.ds(y, RBLK)], disp_v)
        for c in range(C):
            pltpu.sync_copy(im_hbm.at[b, c, pl.ds(y, RBLK)], im_v.at[c])

        def row_body(r, carry2):
            def zero_body(i, c3):
                acc_v[pl.ds(i * L, L)] = jnp.zeros((L,), jnp.float32)
                return c3
            lax.fori_loop(0, 5 * AW // L, zero_body, 0)

            def chunk_body(j, c4):
                d = disp_v[r, pl.ds(j * L, L)]
                gx = lane_f + (j * L).astype(jnp.float32)
                wm = jnp.exp((d - gmin) * LN_BASE)
                t = gx - d + float(PAD)
                xi = t.astype(jnp.int32)
                w1 = t - xi.astype(jnp.float32)
                w0 = 1.0 - w1
                for c in range(C):
                    v = im_v[c, r, pl.ds(j * L, L)] * wm
                    plsc.addupdate_scatter(acc_v, [xi + (c * AW)], v * w0)
                    plsc.addupdate_scatter(acc_v, [xi + (c * AW + 1)], v * w1)
                plsc.addupdate_scatter(acc_v, [xi + (3 * AW)], wm * w0)
                plsc.addupdate_scatter(acc_v, [xi + (3 * AW + 1)], wm * w1)
                plsc.addupdate_scatter(acc_v, [xi + (4 * AW)], w0)
                plsc.addupdate_scatter(acc_v, [xi + (4 * AW + 1)], w1)
                return c4
            lax.fori_loop(0, W // L, chunk_body, 0)

            def fin_body(k, c5):
                m = acc_v[pl.ds(3 * AW + PAD + k * L, L)]
                inv = 1.0 / jnp.maximum(m, EPS)
                for c in range(C):
                    res_v[c, r, pl.ds(k * L, L)] = (
                        acc_v[pl.ds(c * AW + PAD + k * L, L)] * inv)
                o = acc_v[pl.ds(4 * AW + PAD + k * L, L)]
                occ_v[r, pl.ds(k * L, L)] = 1.0 - jnp.minimum(o, 1.0)
                return c5
            lax.fori_loop(0, W // L, fin_body, 0)
            return carry2
        lax.fori_loop(0, RBLK, row_body, 0)

        for c in range(C):
            pltpu.sync_copy(res_v.at[c], res_hbm.at[b, c, pl.ds(y, RBLK)])
        pltpu.sync_copy(occ_v, occ_hbm.at[b, pl.ds(y, RBLK)])
        return carry
    lax.fori_loop(0, NBLK, block_body, 0)


def kernel(im, disp):
    disp3 = disp.reshape(B, H, W)
    gmin = pl.pallas_call(
        _min_body,
        out_shape=jax.ShapeDtypeStruct((8, 128), jnp.float32),
    )(disp.reshape(ROWS, W))

    mesh = plsc.VectorSubcoreMesh(
        core_axis_name="c", subcore_axis_name="s",
        num_cores=NC, num_subcores=NS)
    run = pl.kernel(
        _sc_body,
        out_type=(
            jax.ShapeDtypeStruct((B, C, H, W), jnp.float32),
            jax.ShapeDtypeStruct((B, H, W), jnp.float32),
        ),
        mesh=mesh,
        compiler_params=pltpu.CompilerParams(needs_layout_passes=False),
        scratch_types=[
            pltpu.VMEM((RBLK, W), jnp.float32),
            pltpu.VMEM((C, RBLK, W), jnp.float32),
            pltpu.VMEM((5 * AW,), jnp.float32),
            pltpu.VMEM((C, RBLK, W), jnp.float32),
            pltpu.VMEM((RBLK, W), jnp.float32),
            pltpu.VMEM((128,), jnp.float32),
        ],
    )
    res, occ = run(im, disp3, gmin)
    return res, occ.reshape(B, 1, H, W)


# speedup vs baseline: 137.0819x; 1.4850x over previous
import math

import jax
import jax.numpy as jnp
from jax import lax
from jax.experimental import pallas as pl
from jax.experimental.pallas import tpu as pltpu
from jax.experimental.pallas import tpu_sc as plsc

B, C, H, W = 4, 3, 512, 512
NC, NS, L = 2, 16, 16
NW = NC * NS
ROWS = B * H
RPW = ROWS // NW
TPB = H // RPW
RBLK = 8
NBLK = RPW // RBLK
NBI = NBLK // 2
PAD = 48
AW = 576
EPS = 1e-6
LN_BASE = math.log(1.414)


def _min_body(d_ref, o_ref):
    o_ref[...] = jnp.broadcast_to(jnp.min(d_ref[...]), (8, 128))


def _sc_body(im_hbm, disp_hbm, gmin_hbm, res_hbm, occ_hbm,
             disp_v, im_v, acc_v, res_v, occ_v, gmin_v,
             sem_in0, sem_in1, sem_out0, sem_out1):
    cid = lax.axis_index("c")
    sid = lax.axis_index("s")
    wid = sid * NC + cid
    b = wid // TPB
    y0 = (wid % TPB) * RPW
    sem_in = (sem_in0, sem_in1)
    sem_out = (sem_out0, sem_out1)

    pltpu.sync_copy(gmin_hbm.at[0], gmin_v)
    gmin = gmin_v[pl.ds(0, L)]
    lane_f = lax.iota(jnp.int32, L).astype(jnp.float32)
    ZV = jnp.zeros((L,), jnp.float32)

    def in_copies(s, y):
        cps = [pltpu.make_async_copy(
            disp_hbm.at[b, pl.ds(y, RBLK)], disp_v.at[s], sem_in[s])]
        for c in range(C):
            cps.append(pltpu.make_async_copy(
                im_hbm.at[b, c, pl.ds(y, RBLK)], im_v.at[s, c], sem_in[s]))
        return cps

    def out_copies(s, y):
        cps = []
        for c in range(C):
            cps.append(pltpu.make_async_copy(
                res_v.at[s, c], res_hbm.at[b, c, pl.ds(y, RBLK)], sem_out[s]))
        cps.append(pltpu.make_async_copy(
            occ_v.at[s], occ_hbm.at[b, pl.ds(y, RBLK)], sem_out[s]))
        return cps

    def zero_body(i, c0):
        acc_v[pl.ds(i * L, L)] = ZV
        return c0
    lax.fori_loop(0, 5 * AW // L, zero_body, 0)

    for cp in in_copies(0, y0):
        cp.start()

    def block_pair(bi, carry):
        for h in range(2):
            blk = 2 * bi + h
            y = y0 + blk * RBLK
            s = h
            for cp in in_copies(s, y):
                cp.wait()
            if h == 0:
                for cp in in_copies(1, y + RBLK):
                    cp.start()
            else:
                @pl.when(bi < NBI - 1)
                def _():
                    for cp in in_copies(0, y + RBLK):
                        cp.start()
            @pl.when(bi > 0)
            def _():
                for cp in out_copies(s, y):
                    cp.wait()

            def row_body(r, c2):
                def chunk_body(jj, c4):
                    for u in range(2):
                        j = 2 * jj + u
                        d = disp_v[s, r, pl.ds(j * L, L)]
                        gx = lane_f + (j * L).astype(jnp.float32)
                        wm = jnp.exp((d - gmin) * LN_BASE)
                        t = gx - d + float(PAD)
                        xi = t.astype(jnp.int32)
                        w1 = t - xi.astype(jnp.float32)
                        w0 = 1.0 - w1
                        for c in range(C):
                            v = im_v[s, c, r, pl.ds(j * L, L)] * wm
                            plsc.addupdate_scatter(
                                acc_v, [xi + (c * AW)], v * w0)
                            plsc.addupdate_scatter(
                                acc_v, [xi + (c * AW + 1)], v * w1)
                        plsc.addupdate_scatter(acc_v, [xi + (3 * AW)], wm * w0)
                        plsc.addupdate_scatter(
                            acc_v, [xi + (3 * AW + 1)], wm * w1)
                        plsc.addupdate_scatter(acc_v, [xi + (4 * AW)], w0)
                        plsc.addupdate_scatter(acc_v, [xi + (4 * AW + 1)], w1)
                    return c4
                lax.fori_loop(0, W // (2 * L), chunk_body, 0)

                for c in range(5):
                    for p in range(PAD // L):
                        acc_v[pl.ds(c * AW + p * L, L)] = ZV
                    acc_v[pl.ds(c * AW + PAD + W, L)] = ZV

                def fin_body(k, c5):
                    koff = PAD + k * L
                    m = acc_v[pl.ds(3 * AW + koff, L)]
                    inv = 1.0 / jnp.maximum(m, EPS)
                    for c in range(C):
                        res_v[s, c, r, pl.ds(k * L, L)] = (
                            acc_v[pl.ds(c * AW + koff, L)] * inv)
                        acc_v[pl.ds(c * AW + koff, L)] = ZV
                    o = acc_v[pl.ds(4 * AW + koff, L)]
                    occ_v[s, r, pl.ds(k * L, L)] = 1.0 - jnp.minimum(o, 1.0)
                    acc_v[pl.ds(3 * AW + koff, L)] = ZV
                    acc_v[pl.ds(4 * AW + koff, L)] = ZV
                    return c5
                lax.fori_loop(0, W // L, fin_body, 0)
                return c2
            lax.fori_loop(0, RBLK, row_body, 0)

            for cp in out_copies(s, y):
                cp.start()
        return carry
    lax.fori_loop(0, NBI, block_pair, 0)

    for s in range(2):
        y = y0 + (NBLK - 2 + s) * RBLK
        for cp in out_copies(s, y):
            cp.wait()


def kernel(im, disp):
    disp3 = disp.reshape(B, H, W)
    gmin = pl.pallas_call(
        _min_body,
        out_shape=jax.ShapeDtypeStruct((8, 128), jnp.float32),
    )(disp.reshape(ROWS, W))

    mesh = plsc.VectorSubcoreMesh(
        core_axis_name="c", subcore_axis_name="s",
        num_cores=NC, num_subcores=NS)
    run = pl.kernel(
        _sc_body,
        out_type=(
            jax.ShapeDtypeStruct((B, C, H, W), jnp.float32),
            jax.ShapeDtypeStruct((B, H, W), jnp.float32),
        ),
        mesh=mesh,
        compiler_params=pltpu.CompilerParams(needs_layout_passes=False),
        scratch_types=[
            pltpu.VMEM((2, RBLK, W), jnp.float32),
            pltpu.VMEM((2, C, RBLK, W), jnp.float32),
            pltpu.VMEM((5 * AW,), jnp.float32),
            pltpu.VMEM((2, C, RBLK, W), jnp.float32),
            pltpu.VMEM((2, RBLK, W), jnp.float32),
            pltpu.VMEM((128,), jnp.float32),
            pltpu.SemaphoreType.DMA,
            pltpu.SemaphoreType.DMA,
            pltpu.SemaphoreType.DMA,
            pltpu.SemaphoreType.DMA,
        ],
    )
    res, occ = run(im, disp3, gmin)
    return res, occ.reshape(B, 1, H, W)


# speedup vs baseline: 303.1080x; 2.2111x over previous
import math

import jax
import jax.numpy as jnp
from jax import lax
from jax.experimental import pallas as pl
from jax.experimental.pallas import tpu as pltpu
from jax.experimental.pallas import tpu_sc as plsc

B, C, H, W = 4, 3, 512, 512
NC, NS, L = 2, 16, 16
NW = NC * NS
ROWS = B * H
RPW = ROWS // NW
TPB = H // RPW
RBLK = 8
NBLK = RPW // RBLK
NBI = NBLK // 2
PAD = 48
AW = 576
EPS = 1e-6
LN_BASE = math.log(1.414)


def _min_body(d_ref, o_ref):
    o_ref[...] = jnp.broadcast_to(jnp.min(d_ref[...]), (8, 128))


def _sc_body(im_hbm, disp_hbm, gmin_hbm, res_hbm, occ_hbm,
             disp_v, im_v, acc0, acc1, acc2, acc3, acc4, res_v, occ_v, gmin_v,
             sem_in0, sem_in1, sem_out0, sem_out1):
    accs = (acc0, acc1, acc2, acc3, acc4)
    cid = lax.axis_index("c")
    sid = lax.axis_index("s")
    wid = sid * NC + cid
    b = wid // TPB
    y0 = (wid % TPB) * RPW
    sem_in = (sem_in0, sem_in1)
    sem_out = (sem_out0, sem_out1)

    pltpu.sync_copy(gmin_hbm.at[0], gmin_v)
    gmin = gmin_v[pl.ds(0, L)]
    lane_f = lax.iota(jnp.int32, L).astype(jnp.float32)
    ZV = jnp.zeros((L,), jnp.float32)

    def in_copies(s, y):
        cps = [pltpu.make_async_copy(
            disp_hbm.at[b, pl.ds(y, RBLK)], disp_v.at[s], sem_in[s])]
        for c in range(C):
            cps.append(pltpu.make_async_copy(
                im_hbm.at[b, c, pl.ds(y, RBLK)], im_v.at[s, c], sem_in[s]))
        return cps

    def out_copies(s, y):
        cps = []
        for c in range(C):
            cps.append(pltpu.make_async_copy(
                res_v.at[s, c], res_hbm.at[b, c, pl.ds(y, RBLK)], sem_out[s]))
        cps.append(pltpu.make_async_copy(
            occ_v.at[s], occ_hbm.at[b, pl.ds(y, RBLK)], sem_out[s]))
        return cps

    def zero_body(i, c0):
        for a in accs:
            a[pl.ds(i * L, L)] = ZV
        return c0
    lax.fori_loop(0, AW // L, zero_body, 0)

    for cp in in_copies(0, y0):
        cp.start()

    def block_pair(bi, carry):
        for h in range(2):
            blk = 2 * bi + h
            y = y0 + blk * RBLK
            s = h
            for cp in in_copies(s, y):
                cp.wait()
            if h == 0:
                for cp in in_copies(1, y + RBLK):
                    cp.start()
            else:
                @pl.when(bi < NBI - 1)
                def _():
                    for cp in in_copies(0, y + RBLK):
                        cp.start()
            @pl.when(bi > 0)
            def _():
                for cp in out_copies(s, y):
                    cp.wait()

            def row_body(r, c2):
                @plsc.parallel_loop(0, W // L, unroll=2)
                def chunk_body(j):
                    d = disp_v[s, r, pl.ds(j * L, L)]
                    gx = lane_f + (j * L).astype(jnp.float32)
                    wm = jnp.exp((d - gmin) * LN_BASE)
                    t = gx - d + float(PAD)
                    xi = t.astype(jnp.int32)
                    xj = xi + 1
                    w1 = t - xi.astype(jnp.float32)
                    w0 = 1.0 - w1
                    for c in range(C):
                        v = im_v[s, c, r, pl.ds(j * L, L)] * wm
                        plsc.addupdate_scatter(accs[c], [xi], v * w0)
                        plsc.addupdate_scatter(accs[c], [xj], v * w1)
                    plsc.addupdate_scatter(acc3, [xi], wm * w0)
                    plsc.addupdate_scatter(acc3, [xj], wm * w1)
                    plsc.addupdate_scatter(acc4, [xi], w0)
                    plsc.addupdate_scatter(acc4, [xj], w1)

                for a in accs:
                    for p in range(PAD // L):
                        a[pl.ds(p * L, L)] = ZV
                    a[pl.ds(PAD + W, L)] = ZV

                @plsc.parallel_loop(0, W // L, unroll=2)
                def fin_body(k):
                    koff = PAD + k * L
                    m = acc3[pl.ds(koff, L)]
                    inv = 1.0 / jnp.maximum(m, EPS)
                    for c in range(C):
                        res_v[s, c, r, pl.ds(k * L, L)] = (
                            accs[c][pl.ds(koff, L)] * inv)
                        accs[c][pl.ds(koff, L)] = ZV
                    o = acc4[pl.ds(koff, L)]
                    occ_v[s, r, pl.ds(k * L, L)] = 1.0 - jnp.minimum(o, 1.0)
                    acc3[pl.ds(koff, L)] = ZV
                    acc4[pl.ds(koff, L)] = ZV
                return c2
            lax.fori_loop(0, RBLK, row_body, 0)

            for cp in out_copies(s, y):
                cp.start()
        return carry
    lax.fori_loop(0, NBI, block_pair, 0)

    for s in range(2):
        y = y0 + (NBLK - 2 + s) * RBLK
        for cp in out_copies(s, y):
            cp.wait()


def kernel(im, disp):
    disp3 = disp.reshape(B, H, W)
    gmin = pl.pallas_call(
        _min_body,
        out_shape=jax.ShapeDtypeStruct((8, 128), jnp.float32),
    )(disp.reshape(ROWS, W))

    mesh = plsc.VectorSubcoreMesh(
        core_axis_name="c", subcore_axis_name="s",
        num_cores=NC, num_subcores=NS)
    run = pl.kernel(
        _sc_body,
        out_type=(
            jax.ShapeDtypeStruct((B, C, H, W), jnp.float32),
            jax.ShapeDtypeStruct((B, H, W), jnp.float32),
        ),
        mesh=mesh,
        compiler_params=pltpu.CompilerParams(needs_layout_passes=False),
        scratch_types=[
            pltpu.VMEM((2, RBLK, W), jnp.float32),
            pltpu.VMEM((2, C, RBLK, W), jnp.float32),
            pltpu.VMEM((AW,), jnp.float32),
            pltpu.VMEM((AW,), jnp.float32),
            pltpu.VMEM((AW,), jnp.float32),
            pltpu.VMEM((AW,), jnp.float32),
            pltpu.VMEM((AW,), jnp.float32),
            pltpu.VMEM((2, C, RBLK, W), jnp.float32),
            pltpu.VMEM((2, RBLK, W), jnp.float32),
            pltpu.VMEM((128,), jnp.float32),
            pltpu.SemaphoreType.DMA,
            pltpu.SemaphoreType.DMA,
            pltpu.SemaphoreType.DMA,
            pltpu.SemaphoreType.DMA,
        ],
    )
    res, occ = run(im, disp3, gmin)
    return res, occ.reshape(B, 1, H, W)


# speedup vs baseline: 311.6267x; 1.0281x over previous
import math

import jax
import jax.numpy as jnp
from jax import lax
from jax.experimental import pallas as pl
from jax.experimental.pallas import tpu as pltpu
from jax.experimental.pallas import tpu_sc as plsc

B, C, H, W = 4, 3, 512, 512
NC, NS, L = 2, 16, 16
NW = NC * NS
ROWS = B * H
RPW = ROWS // NW
TPB = H // RPW
RBLK = 8
NBLK = RPW // RBLK
NBI = NBLK // 2
PAD = 48
AW = 576
EPS = 1e-6
LN_BASE = math.log(1.414)


def _min_body(d_ref, o_ref):
    o_ref[...] = jnp.broadcast_to(jnp.min(d_ref[...]), (8, 128))


def _sc_body(im_hbm, disp_hbm, gmin_hbm, res_hbm, occ_hbm,
             disp_v, im_v, acc0, acc1, acc2, acc3, acc4, res_v, occ_v, gmin_v,
             sem_in0, sem_in1, sem_out0, sem_out1):
    accs = (acc0, acc1, acc2, acc3, acc4)
    cid = lax.axis_index("c")
    sid = lax.axis_index("s")
    wid = sid * NC + cid
    b = wid // TPB
    y0 = (wid % TPB) * RPW
    sem_in = (sem_in0, sem_in1)
    sem_out = (sem_out0, sem_out1)

    pltpu.sync_copy(gmin_hbm.at[0], gmin_v)
    gmin = gmin_v[pl.ds(0, L)]
    lane_f = lax.iota(jnp.int32, L).astype(jnp.float32)
    ZV = jnp.zeros((L,), jnp.float32)

    def in_copies(s, y):
        cps = [pltpu.make_async_copy(
            disp_hbm.at[b, pl.ds(y, RBLK)], disp_v.at[s], sem_in[s])]
        for c in range(C):
            cps.append(pltpu.make_async_copy(
                im_hbm.at[b, c, pl.ds(y, RBLK)], im_v.at[s, c], sem_in[s]))
        return cps

    def out_copies(s, y):
        cps = []
        for c in range(C):
            cps.append(pltpu.make_async_copy(
                res_v.at[s, c], res_hbm.at[b, c, pl.ds(y, RBLK)], sem_out[s]))
        cps.append(pltpu.make_async_copy(
            occ_v.at[s], occ_hbm.at[b, pl.ds(y, RBLK)], sem_out[s]))
        return cps

    def zero_body(i, c0):
        for a in accs:
            a[pl.ds(i * L, L)] = ZV
        return c0
    lax.fori_loop(0, RBLK * AW // L, zero_body, 0)

    for cp in in_copies(0, y0):
        cp.start()

    def block_pair(bi, carry):
        for h in range(2):
            blk = 2 * bi + h
            y = y0 + blk * RBLK
            s = h
            for cp in in_copies(s, y):
                cp.wait()
            if h == 0:
                for cp in in_copies(1, y + RBLK):
                    cp.start()
            else:
                @pl.when(bi < NBI - 1)
                def _():
                    for cp in in_copies(0, y + RBLK):
                        cp.start()
            @pl.when(bi > 0)
            def _():
                for cp in out_copies(s, y):
                    cp.wait()

            @plsc.parallel_loop(0, RBLK * (W // L), unroll=2)
            def chunk_body(i):
                r = i >> 5
                j = i & (W // L - 1)
                base = r * AW
                d = disp_v[s, r, pl.ds(j * L, L)]
                gx = lane_f + (j * L).astype(jnp.float32)
                wm = jnp.exp((d - gmin) * LN_BASE)
                t = gx - d + float(PAD)
                xt = t.astype(jnp.int32)
                w1 = t - xt.astype(jnp.float32)
                w0 = 1.0 - w1
                xi = xt + base
                xj = xi + 1
                for c in range(C):
                    v = im_v[s, c, r, pl.ds(j * L, L)] * wm
                    plsc.addupdate_scatter(accs[c], [xi], v * w0)
                    plsc.addupdate_scatter(accs[c], [xj], v * w1)
                plsc.addupdate_scatter(acc3, [xi], wm * w0)
                plsc.addupdate_scatter(acc3, [xj], wm * w1)
                plsc.addupdate_scatter(acc4, [xi], w0)
                plsc.addupdate_scatter(acc4, [xj], w1)

            @plsc.parallel_loop(0, RBLK)
            def pad_body(r):
                base = r * AW
                for a in accs:
                    for p in range(PAD // L):
                        a[pl.ds(base + p * L, L)] = ZV
                    a[pl.ds(base + PAD + W, L)] = ZV

            @plsc.parallel_loop(0, RBLK * (W // L), unroll=2)
            def fin_body(i):
                r = i >> 5
                k = i & (W // L - 1)
                koff = r * AW + PAD + k * L
                m = acc3[pl.ds(koff, L)]
                inv = 1.0 / jnp.maximum(m, EPS)
                for c in range(C):
                    res_v[s, c, r, pl.ds(k * L, L)] = (
                        accs[c][pl.ds(koff, L)] * inv)
                    accs[c][pl.ds(koff, L)] = ZV
                o = acc4[pl.ds(koff, L)]
                occ_v[s, r, pl.ds(k * L, L)] = 1.0 - jnp.minimum(o, 1.0)
                acc3[pl.ds(koff, L)] = ZV
                acc4[pl.ds(koff, L)] = ZV

            for cp in out_copies(s, y):
                cp.start()
        return carry
    lax.fori_loop(0, NBI, block_pair, 0)

    for s in range(2):
        y = y0 + (NBLK - 2 + s) * RBLK
        for cp in out_copies(s, y):
            cp.wait()


def kernel(im, disp):
    disp3 = disp.reshape(B, H, W)
    gmin = pl.pallas_call(
        _min_body,
        out_shape=jax.ShapeDtypeStruct((8, 128), jnp.float32),
    )(disp.reshape(ROWS, W))

    mesh = plsc.VectorSubcoreMesh(
        core_axis_name="c", subcore_axis_name="s",
        num_cores=NC, num_subcores=NS)
    run = pl.kernel(
        _sc_body,
        out_type=(
            jax.ShapeDtypeStruct((B, C, H, W), jnp.float32),
            jax.ShapeDtypeStruct((B, H, W), jnp.float32),
        ),
        mesh=mesh,
        compiler_params=pltpu.CompilerParams(needs_layout_passes=False),
        scratch_types=[
            pltpu.VMEM((2, RBLK, W), jnp.float32),
            pltpu.VMEM((2, C, RBLK, W), jnp.float32),
            pltpu.VMEM((RBLK * AW,), jnp.float32),
            pltpu.VMEM((RBLK * AW,), jnp.float32),
            pltpu.VMEM((RBLK * AW,), jnp.float32),
            pltpu.VMEM((RBLK * AW,), jnp.float32),
            pltpu.VMEM((RBLK * AW,), jnp.float32),
            pltpu.VMEM((2, C, RBLK, W), jnp.float32),
            pltpu.VMEM((2, RBLK, W), jnp.float32),
            pltpu.VMEM((128,), jnp.float32),
            pltpu.SemaphoreType.DMA,
            pltpu.SemaphoreType.DMA,
            pltpu.SemaphoreType.DMA,
            pltpu.SemaphoreType.DMA,
        ],
    )
    res, occ = run(im, disp3, gmin)
    return res, occ.reshape(B, 1, H, W)
